# baseline (speedup 1.0000x reference)
import functools

import jax
import jax.numpy as jnp
from jax import lax
from jax.experimental import pallas as pl
from jax.experimental.pallas import tpu as pltpu
from jax.experimental.pallas import tpu_sc as plsc

N = 10000
NNP = 10112
E0 = 320000
E = 331776
NW = 32
PW = E // NW
K12 = 3456
PT = E // 16
K3 = 288

F32 = jnp.float32
I32 = jnp.int32

_MESH = dict(core_axis_name="c", subcore_axis_name="s", num_cores=2,
             num_subcores=16)


def _sc_mesh():
    return plsc.VectorSubcoreMesh(**_MESH)


def _e1_body(hn, src_h, dst_h, asn_h, adn_h, ninf_h, alpha_h, mpart_h,
             asn_sp, adn_sp, m_v, srcv, dstv, aidxb, bidxb, ag, bg, ab):
    c = lax.axis_index("c")
    s = lax.axis_index("s")
    wid = s * 2 + c
    sh = hn.bit_length() - 1
    msk = hn - 1
    kf = K12 * hn

    @pl.when(s == 0)
    def _():
        pltpu.sync_copy(asn_h, asn_sp)
        pltpu.sync_copy(adn_h, adn_sp)

    plsc.subcore_barrier()
    pltpu.sync_copy(ninf_h, m_v)
    lanes = lax.iota(I32, 16)

    for k in range(PW // K12):
        base = wid * PW + k * K12
        pltpu.sync_copy(src_h.at[pl.ds(base, K12)], srcv)
        pltpu.sync_copy(dst_h.at[pl.ds(base, K12)], dstv)

        def idxbody(j, _):
            f = j * 16 + lanes
            e_of = lax.shift_right_logical(f, sh)
            h_of = lax.bitwise_and(f, msk)
            sv = plsc.load_gather(srcv, [e_of])
            dv = plsc.load_gather(dstv, [e_of])
            aidxb[pl.ds(j * 16, 16)] = sv * hn + h_of
            bidxb[pl.ds(j * 16, 16)] = dv * hn + h_of
            return 0

        lax.fori_loop(0, kf // 16, idxbody, 0)
        pltpu.sync_copy(asn_sp.at[aidxb], ag)
        pltpu.sync_copy(adn_sp.at[bidxb], bg)

        def jbody(j, _):
            al = ag[pl.ds(j * 16, 16)] + bg[pl.ds(j * 16, 16)]
            al = jnp.where(al >= 0.0, al, 0.2 * al)
            ab[pl.ds(j * 16, 16)] = al
            midx = bidxb[pl.ds(j * 16, 16)]
            cur = plsc.load_gather(m_v, [midx])

            def wcond(cu):
                return jnp.any(al > cu)

            def wbody(cu):
                plsc.store_scatter(m_v, [midx], al, mask=al > cu)
                return plsc.load_gather(m_v, [midx])

            lax.while_loop(wcond, wbody, cur)
            return 0

        lax.fori_loop(0, kf // 16, jbody, 0)
        pltpu.sync_copy(ab, alpha_h.at[pl.ds(base * hn, kf)])

    pltpu.sync_copy(m_v, mpart_h.at[pl.ds(wid * (NNP * hn), NNP * hn)])


def _make_e1(hn):
    mw = NNP * hn
    kf = K12 * hn
    return pl.kernel(
        functools.partial(_e1_body, hn),
        out_type=(jax.ShapeDtypeStruct((E * hn,), F32),
                  jax.ShapeDtypeStruct((NW * mw,), F32)),
        mesh=_sc_mesh(),
        compiler_params=pltpu.CompilerParams(needs_layout_passes=False),
        scratch_types=[
            pltpu.VMEM_SHARED((mw,), F32),
            pltpu.VMEM_SHARED((mw,), F32),
            pltpu.VMEM((mw,), F32),
            pltpu.VMEM((K12,), I32),
            pltpu.VMEM((K12,), I32),
            pltpu.VMEM((kf,), I32),
            pltpu.VMEM((kf,), I32),
            pltpu.VMEM((kf,), F32),
            pltpu.VMEM((kf,), F32),
            pltpu.VMEM((kf,), F32),
        ],
    )


def _e2_body(hn, dst_h, alpha_h, m_h, zs_h, e_h, spart_h,
             m_sp, s_sp, dstv, bidxb, avb, mg, eb):
    c = lax.axis_index("c")
    s = lax.axis_index("s")
    wid = s * 2 + c
    sh = hn.bit_length() - 1
    msk = hn - 1
    kf = K12 * hn

    @pl.when(s == 0)
    def _():
        pltpu.sync_copy(m_h, m_sp)
        pltpu.sync_copy(zs_h, s_sp)

    plsc.subcore_barrier()
    lanes = lax.iota(I32, 16)

    for k in range(PW // K12):
        base = wid * PW + k * K12
        pltpu.sync_copy(dst_h.at[pl.ds(base, K12)], dstv)
        pltpu.sync_copy(alpha_h.at[pl.ds(base * hn, kf)], avb)

        def idxbody(j, _):
            f = j * 16 + lanes
            e_of = lax.shift_right_logical(f, sh)
            h_of = lax.bitwise_and(f, msk)
            dv = plsc.load_gather(dstv, [e_of])
            bidxb[pl.ds(j * 16, 16)] = dv * hn + h_of
            return 0

        lax.fori_loop(0, kf // 16, idxbody, 0)
        pltpu.sync_copy(m_sp.at[bidxb], mg)

        def jbody(j, _):
            ev = jnp.exp(avb[pl.ds(j * 16, 16)] - mg[pl.ds(j * 16, 16)])
            eb[pl.ds(j * 16, 16)] = ev
            return 0

        lax.fori_loop(0, kf // 16, jbody, 0)
        pltpu.sync_copy(eb, e_h.at[pl.ds(base * hn, kf)])
        pltpu.sync_copy(eb, s_sp.at[bidxb], add=True)

    plsc.subcore_barrier()

    @pl.when(s == 0)
    def _():
        pltpu.sync_copy(s_sp, spart_h.at[pl.ds(c * (NNP * hn), NNP * hn)])


def _make_e2(hn):
    mw = NNP * hn
    kf = K12 * hn
    return pl.kernel(
        functools.partial(_e2_body, hn),
        out_type=(jax.ShapeDtypeStruct((E * hn,), F32),
                  jax.ShapeDtypeStruct((2 * mw,), F32)),
        mesh=_sc_mesh(),
        compiler_params=pltpu.CompilerParams(needs_layout_passes=False),
        scratch_types=[
            pltpu.VMEM_SHARED((mw,), F32),
            pltpu.VMEM_SHARED((mw,), F32),
            pltpu.VMEM((K12,), I32),
            pltpu.VMEM((kf,), I32),
            pltpu.VMEM((kf,), F32),
            pltpu.VMEM((kf,), F32),
            pltpu.VMEM((kf,), F32),
        ],
    )


def _e3_body(hn, hph, fh, src_h, dst_h, e_h, hh_h, zu_h, u_h,
             u_sp, srcv, dstv, eb, rows_v, srcv2, dstv2, eb2, rows2,
             gsa, gsb, ssa):
    c = lax.axis_index("c")
    s = lax.axis_index("s")
    esplit = hn == 1
    k3 = 128
    nchunk = (E // NW) // k3 if esplit else PT // k3

    @pl.when(s == 0)
    def _():
        pltpu.sync_copy(zu_h, u_sp)

    plsc.subcore_barrier()

    def chunk_base(k):
        if esplit:
            return (s * 2 + c) * (E // NW) + k * k3
        return s * PT + k * k3

    def load_idx(k, sv, dv, ev):
        base = chunk_base(k)
        pltpu.sync_copy(src_h.at[pl.ds(base, k3)], sv)
        pltpu.sync_copy(dst_h.at[pl.ds(base, k3)], dv)
        pltpu.sync_copy(e_h.at[pl.ds(base * hn, k3 * hn)], ev)

    def start_gather(sv, rv, sem):
        if esplit:
            return pltpu.async_copy(hh_h.at[sv], rv, sem)
        return pltpu.async_copy(hh_h.at[c].at[sv], rv, sem)

    def mul(rv, ev):
        epq = 16 // hn

        def ibody(q, _):
            evec = ev[pl.ds(q * 16, 16)]
            for r in range(epq):
                i = q * epq + r
                for hh in range(hph):
                    if hn > 1:
                        w = jnp.where(c == 0, evec[r * hn + hh],
                                      evec[r * hn + hph + hh])
                    else:
                        w = evec[r]
                    for g in range(fh // (hph * 16)):
                        off = hh * (fh // hph) + g * 16
                        v = rv[i, pl.ds(off, 16)]
                        rv[i, pl.ds(off, 16)] = v * w
            return 0

        lax.fori_loop(0, k3 // epq, ibody, 0)

    def wait_gather(sv, rv, sem):
        if esplit:
            pltpu.make_async_copy(hh_h.at[sv], rv, sem).wait()
        else:
            pltpu.make_async_copy(hh_h.at[c].at[sv], rv, sem).wait()

    def wait_scatter(rv, dv, sem):
        pltpu.make_async_copy(rv, u_sp.at[dv], sem).wait()

    npair = nchunk // 2

    load_idx(0, srcv, dstv, eb)
    start_gather(srcv, rows_v, gsa)

    def pbody(t, _):
        wait_gather(srcv, rows_v, gsa)
        mul(rows_v, eb)

        @pl.when(t > 0)
        def _():
            wait_scatter(rows2, dstv2, gsb)

        load_idx(2 * t + 1, srcv2, dstv2, eb2)
        start_gather(srcv2, rows2, ssa)
        pltpu.async_copy(rows_v, u_sp.at[dstv], gsa, add=True)

        wait_gather(srcv2, rows2, ssa)
        mul(rows2, eb2)
        wait_scatter(rows_v, dstv, gsa)

        @pl.when(t + 1 < npair)
        def _():
            load_idx(2 * t + 2, srcv, dstv, eb)
            start_gather(srcv, rows_v, gsa)

        pltpu.async_copy(rows2, u_sp.at[dstv2], gsb, add=True)
        return 0

    lax.fori_loop(0, npair, pbody, 0)
    wait_scatter(rows2, dstv2, gsb)

    if nchunk % 2:
        load_idx(nchunk - 1, srcv, dstv, eb)
        start_gather(srcv, rows_v, gsa)
        wait_gather(srcv, rows_v, gsa)
        mul(rows_v, eb)
        pltpu.sync_copy(rows_v, u_sp.at[dstv], add=True)

    plsc.subcore_barrier()
    nr = NNP // 16
    pltpu.sync_copy(u_sp.at[pl.ds(s * nr, nr)], u_h.at[c, pl.ds(s * nr, nr)])


def _make_e3(hn, hph, fh):
    k3 = 128
    return pl.kernel(
        functools.partial(_e3_body, hn, hph, fh),
        out_type=jax.ShapeDtypeStruct((2, NNP, fh), F32),
        mesh=_sc_mesh(),
        compiler_params=pltpu.CompilerParams(needs_layout_passes=False),
        scratch_types=[
            pltpu.VMEM_SHARED((NNP, fh), F32),
            pltpu.VMEM((k3,), I32),
            pltpu.VMEM((k3,), I32),
            pltpu.VMEM((k3 * hn,), F32),
            pltpu.VMEM((k3, fh), F32),
            pltpu.VMEM((k3,), I32),
            pltpu.VMEM((k3,), I32),
            pltpu.VMEM((k3 * hn,), F32),
            pltpu.VMEM((k3, fh), F32),
            pltpu.SemaphoreType.DMA,
            pltpu.SemaphoreType.DMA,
            pltpu.SemaphoreType.DMA,
        ],
    )


def _d1_body(nin, split, x_refs, w_ref, as_ref, ad_ref, h2_ref, asn_ref,
             adn_ref):
    x = x_refs[0][...]
    for r in x_refs[1:]:
        x = x + r[...]
    h = jnp.dot(x, w_ref[...], preferred_element_type=F32)
    if split:
        fh = h2_ref.shape[2]
        h2_ref[0] = h[:, :fh]
        h2_ref[1] = h[:, fh:]
    else:
        h2_ref[...] = h
    asn_ref[...] = jnp.dot(h, as_ref[...], preferred_element_type=F32)
    adn_ref[...] = jnp.dot(h, ad_ref[...], preferred_element_type=F32)


def _d1(xs, wp, asb, adb, fh, hn):
    nin = len(xs)
    split = hn > 1
    hshape = (2, NNP, fh) if split else (NNP, 2 * fh)

    def body(*refs):
        _d1_body(nin, split, refs[:nin], *refs[nin:])

    return pl.pallas_call(
        body,
        out_shape=(jax.ShapeDtypeStruct(hshape, F32),
                   jax.ShapeDtypeStruct((NNP, hn), F32),
                   jax.ShapeDtypeStruct((NNP, hn), F32)),
    )(*xs, wp, asb, adb)


def _d2_body(mp_ref, m_ref):
    m = jnp.max(mp_ref[...], axis=0)
    m_ref[...] = jnp.where(jnp.isfinite(m), m, 0.0)


def _d2(mpart):
    mw = mpart.shape[1]
    return pl.pallas_call(
        _d2_body,
        out_shape=jax.ShapeDtypeStruct((mw,), F32),
    )(mpart)


def _d3_body(hn, fo, sp_ref, u_ref, b_ref, g_ref, bb_ref, o_ref):
    hph = max(hn // 2, 1)
    chn = fo // hn if hn > 1 else fo
    spn = sp_ref[0] + sp_ref[1]
    if hn == 1:
        xarr = (u_ref[0] + u_ref[1]) / (spn[:, 0:1] + 1e-16)
    else:
        blocks = []
        for c in range(2):
            for hh in range(hph):
                hg = c * hph + hh
                sh = spn[:, hg:hg + 1]
                blk = u_ref[c, :, pl.ds(hh * chn, chn)]
                blocks.append(blk / (sh + 1e-16))
        xarr = jnp.concatenate(blocks, axis=1)
    xb = xarr + b_ref[...]
    valid = lax.broadcasted_iota(I32, (NNP, 1), 0) < N
    xm = jnp.where(valid, xb, 0.0)
    mu = jnp.sum(xm, axis=0, keepdims=True) / N
    d = xb - mu
    var = jnp.sum(jnp.where(valid, d * d, 0.0), axis=0, keepdims=True) / N
    y = g_ref[...] * d / jnp.sqrt(var + 1e-5) + bb_ref[...]
    o_ref[...] = jnp.maximum(y, 0.0)


def _d3(spart, u, b, g, bb, hn, fo):
    sp2 = spart.reshape(2, NNP, hn)
    return pl.pallas_call(
        functools.partial(_d3_body, hn, fo),
        out_shape=jax.ShapeDtypeStruct((NNP, fo), F32),
    )(sp2, u, b.reshape(1, -1), g.reshape(1, -1), bb.reshape(1, -1))


def _pool_body(x4_ref, batch_ref, gf_ref, wg1_ref, bg1_ref, wg2_ref, bg2_ref,
               wp_ref, bp_ref, o_ref):
    x4 = x4_ref[...]
    gate1 = jnp.maximum(
        jnp.dot(x4, wg1_ref[...], preferred_element_type=F32)
        + bg1_ref[...], 0.0)
    gate8 = jnp.dot(gate1, wg2_ref[...], preferred_element_type=F32)
    gate = gate8[:, :1] + bg2_ref[...]
    bcol = batch_ref[...].reshape(N, 1)
    gids = lax.broadcasted_iota(I32, (1, 64), 1)
    onehot = bcol == gids
    gm = jnp.max(jnp.where(onehot, gate, -jnp.inf), axis=0)
    gm = jnp.where(jnp.isfinite(gm), gm, 0.0)
    e64 = jnp.where(onehot, jnp.exp(gate - gm[None, :]), 0.0)
    s64 = jnp.sum(e64, axis=0)
    emb = lax.dot_general(e64, x4, (((0,), (0,)), ((), ())),
                          preferred_element_type=F32)
    emb = emb / (s64[:, None] + 1e-16)
    gf = jnp.maximum(
        jnp.dot(gf_ref[...], wp_ref[...], preferred_element_type=F32)
        + bp_ref[...], 0.0)
    o_ref[:, :128] = emb
    o_ref[:, 128:] = gf


def _pool(x4, batch2d, gfp, wg1, bg1, wg2p, bg2, wpp, bp):
    return pl.pallas_call(
        _pool_body,
        out_shape=jax.ShapeDtypeStruct((64, 160), F32),
    )(x4, batch2d, gfp, wg1, bg1, wg2p, bg2, wpp, bp)


def _blockdiag(a, hn, chn, f):
    out = jnp.zeros((f, hn), F32)
    for h in range(hn):
        out = out.at[h * chn:(h + 1) * chn, h].set(a[h])
    return out


def _layer(x_list, src, dst, ninf4, ninf1, zs4, zs1, zu128, zu64, params, l,
           hn, chn):
    f = hn * chn
    fh = f // 2
    fh3 = f if hn == 1 else f // 2
    hph = max(hn // 2, 1)
    din = params['W%d' % l].shape[0]
    din_p = ((din + 63) // 64) * 64
    wp = jnp.zeros((din_p, f), F32).at[:din].set(params['W%d' % l])
    asb = _blockdiag(params['as%d' % l], hn, chn, f)
    adb = _blockdiag(params['ad%d' % l], hn, chn, f)

    h2, asn, adn = _d1(x_list, wp, asb, adb, fh, hn)
    ninf = ninf4 if hn == 4 else ninf1
    zs = zs4 if hn == 4 else zs1
    zu = zu128
    alpha, mpart = _make_e1(hn)(src, dst, asn.reshape(-1), adn.reshape(-1),
                                ninf)
    m = _d2(mpart.reshape(NW, -1))
    ev, spart = _make_e2(hn)(dst, alpha, m, zs)
    u = _make_e3(hn, hph, fh3)(src, dst, ev, h2, zu)
    xn = _d3(spart, u, params['b%d' % l], params['bng%d' % l],
             params['bnb%d' % l], hn, f)
    return xn


def kernel(x, edge_index, batch, global_features, params):
    loop = jnp.arange(N, dtype=I32)
    padv = jnp.full((E - E0 - N,), N, I32)
    src = jnp.concatenate([edge_index[0], loop, padv])
    dst = jnp.concatenate([edge_index[1], loop, padv])

    ninf4 = jnp.full((NNP * 4,), -3.0e38, F32)
    ninf1 = jnp.full((NNP,), -3.0e38, F32)
    zs4 = jnp.zeros((NNP * 4,), F32)
    zs1 = jnp.zeros((NNP,), F32)
    zu128 = jnp.zeros((NNP, 128), F32)
    zu64 = jnp.zeros((NNP, 64), F32)

    x0 = jnp.zeros((NNP, 64), F32).at[:N, :58].set(x)
    args = (src, dst, ninf4, ninf1, zs4, zs1, zu128, zu64, params)
    x1 = _layer([x0], *args, 0, 4, 64)
    x2 = _layer([x1], *args, 1, 4, 64)
    x3 = _layer([x1, x2], *args, 2, 4, 64)
    x4 = _layer([x3], *args, 3, 1, 128)

    x4s = x4[:N]
    batch2d = batch.reshape(N, 1)
    gfp = jnp.zeros((64, 8), F32).at[:, :7].set(global_features)
    wg2p = jnp.zeros((32, 8), F32).at[:, 0].set(params['Wg2'][:, 0])
    wpp = jnp.zeros((8, 32), F32).at[:7].set(params['Wp'])
    out = _pool(x4s, batch2d, gfp, params['Wg1'],
                params['bg1'].reshape(1, -1), wg2p,
                params['bg2'].reshape(1, 1), wpp,
                params['bp'].reshape(1, -1))
    return out


# speedup vs baseline: 1.1270x; 1.1270x over previous
import functools

import jax
import jax.numpy as jnp
from jax import lax
from jax.experimental import pallas as pl
from jax.experimental.pallas import tpu as pltpu
from jax.experimental.pallas import tpu_sc as plsc

N = 10000
NNP = 10112
E0 = 320000
E = 331776
NW = 32
PW = E // NW
K12 = 3456
PT = E // 16
K3 = 288

F32 = jnp.float32
I32 = jnp.int32

_MESH = dict(core_axis_name="c", subcore_axis_name="s", num_cores=2,
             num_subcores=16)


def _sc_mesh():
    return plsc.VectorSubcoreMesh(**_MESH)


def _e1_body(hn, src_h, dst_h, asn_h, adn_h, ninf_h, alpha_h, mpart_h,
             asn_sp, adn_sp, m_v, srcv, dstv, aidxb, bidxb, ag, bg, ab,
             gs1, gs2):
    c = lax.axis_index("c")
    s = lax.axis_index("s")
    wid = s * 2 + c
    sh = hn.bit_length() - 1
    msk = hn - 1
    kf = K12 * hn

    @pl.when(s == 0)
    def _():
        pltpu.sync_copy(asn_h, asn_sp)
        pltpu.sync_copy(adn_h, adn_sp)

    plsc.subcore_barrier()
    pltpu.sync_copy(ninf_h, m_v)
    lanes = lax.iota(I32, 16)

    for k in range(PW // K12):
        base = wid * PW + k * K12
        pltpu.sync_copy(src_h.at[pl.ds(base, K12)], srcv)
        pltpu.sync_copy(dst_h.at[pl.ds(base, K12)], dstv)

        def idxbody(j, _):
            f = j * 16 + lanes
            e_of = lax.shift_right_logical(f, sh)
            h_of = lax.bitwise_and(f, msk)
            sv = plsc.load_gather(srcv, [e_of])
            dv = plsc.load_gather(dstv, [e_of])
            aidxb[pl.ds(j * 16, 16)] = sv * hn + h_of
            bidxb[pl.ds(j * 16, 16)] = dv * hn + h_of
            return 0

        lax.fori_loop(0, kf // 16, idxbody, 0)
        dga = pltpu.async_copy(asn_sp.at[aidxb], ag, gs1)
        dgb = pltpu.async_copy(adn_sp.at[bidxb], bg, gs2)
        dga.wait()
        dgb.wait()

        def jbody(j, _):
            al = ag[pl.ds(j * 16, 16)] + bg[pl.ds(j * 16, 16)]
            al = jnp.where(al >= 0.0, al, 0.2 * al)
            ab[pl.ds(j * 16, 16)] = al
            midx = bidxb[pl.ds(j * 16, 16)]
            cur = plsc.load_gather(m_v, [midx])

            def wcond(cu):
                return jnp.any(al > cu)

            def wbody(cu):
                plsc.store_scatter(m_v, [midx], al, mask=al > cu)
                return plsc.load_gather(m_v, [midx])

            lax.while_loop(wcond, wbody, cur)
            return 0

        lax.fori_loop(0, kf // 16, jbody, 0)
        pltpu.sync_copy(ab, alpha_h.at[pl.ds(base * hn, kf)])

    pltpu.sync_copy(m_v, mpart_h.at[pl.ds(wid * (NNP * hn), NNP * hn)])


def _make_e1(hn):
    mw = NNP * hn
    kf = K12 * hn
    return pl.kernel(
        functools.partial(_e1_body, hn),
        out_type=(jax.ShapeDtypeStruct((E * hn,), F32),
                  jax.ShapeDtypeStruct((NW * mw,), F32)),
        mesh=_sc_mesh(),
        compiler_params=pltpu.CompilerParams(needs_layout_passes=False),
        scratch_types=[
            pltpu.VMEM_SHARED((mw,), F32),
            pltpu.VMEM_SHARED((mw,), F32),
            pltpu.VMEM((mw,), F32),
            pltpu.VMEM((K12,), I32),
            pltpu.VMEM((K12,), I32),
            pltpu.VMEM((kf,), I32),
            pltpu.VMEM((kf,), I32),
            pltpu.VMEM((kf,), F32),
            pltpu.VMEM((kf,), F32),
            pltpu.VMEM((kf,), F32),
            pltpu.SemaphoreType.DMA,
            pltpu.SemaphoreType.DMA,
        ],
    )


def _e2_body(hn, dst_h, alpha_h, m_h, zs_h, e_h, spart_h,
             m_sp, s_sp, dstv, bidxb, avb, mg, eb, gs1):
    c = lax.axis_index("c")
    s = lax.axis_index("s")
    wid = s * 2 + c
    sh = hn.bit_length() - 1
    msk = hn - 1
    kf = K12 * hn

    @pl.when(s == 0)
    def _():
        pltpu.sync_copy(m_h, m_sp)
        pltpu.sync_copy(zs_h, s_sp)

    plsc.subcore_barrier()
    lanes = lax.iota(I32, 16)

    for k in range(PW // K12):
        base = wid * PW + k * K12
        pltpu.sync_copy(dst_h.at[pl.ds(base, K12)], dstv)
        dav = pltpu.async_copy(alpha_h.at[pl.ds(base * hn, kf)], avb, gs1)

        def idxbody(j, _):
            f = j * 16 + lanes
            e_of = lax.shift_right_logical(f, sh)
            h_of = lax.bitwise_and(f, msk)
            dv = plsc.load_gather(dstv, [e_of])
            bidxb[pl.ds(j * 16, 16)] = dv * hn + h_of
            return 0

        lax.fori_loop(0, kf // 16, idxbody, 0)
        pltpu.sync_copy(m_sp.at[bidxb], mg)
        dav.wait()

        def jbody(j, _):
            ev = jnp.exp(avb[pl.ds(j * 16, 16)] - mg[pl.ds(j * 16, 16)])
            eb[pl.ds(j * 16, 16)] = ev
            return 0

        lax.fori_loop(0, kf // 16, jbody, 0)
        pltpu.sync_copy(eb, e_h.at[pl.ds(base * hn, kf)])
        pltpu.sync_copy(eb, s_sp.at[bidxb], add=True)

    plsc.subcore_barrier()

    @pl.when(s == 0)
    def _():
        pltpu.sync_copy(s_sp, spart_h.at[pl.ds(c * (NNP * hn), NNP * hn)])


def _make_e2(hn):
    mw = NNP * hn
    kf = K12 * hn
    return pl.kernel(
        functools.partial(_e2_body, hn),
        out_type=(jax.ShapeDtypeStruct((E * hn,), F32),
                  jax.ShapeDtypeStruct((2 * mw,), F32)),
        mesh=_sc_mesh(),
        compiler_params=pltpu.CompilerParams(needs_layout_passes=False),
        scratch_types=[
            pltpu.VMEM_SHARED((mw,), F32),
            pltpu.VMEM_SHARED((mw,), F32),
            pltpu.VMEM((K12,), I32),
            pltpu.VMEM((kf,), I32),
            pltpu.VMEM((kf,), F32),
            pltpu.VMEM((kf,), F32),
            pltpu.VMEM((kf,), F32),
            pltpu.SemaphoreType.DMA,
        ],
    )


def _e3_body(hn, hph, fh, src_h, dst_h, e_h, hh_h, zu_h, u_h,
             u_sp, srcv, dstv, eb, rows_v, srcv2, dstv2, eb2, rows2,
             gsa, gsb, ssa):
    c = lax.axis_index("c")
    s = lax.axis_index("s")
    esplit = hn == 1
    k3 = 128
    nchunk = (E // NW) // k3 if esplit else PT // k3

    @pl.when(s == 0)
    def _():
        pltpu.sync_copy(zu_h, u_sp)

    plsc.subcore_barrier()

    def chunk_base(k):
        if esplit:
            return (s * 2 + c) * (E // NW) + k * k3
        return s * PT + k * k3

    def load_idx(k, sv, dv, ev):
        base = chunk_base(k)
        pltpu.sync_copy(src_h.at[pl.ds(base, k3)], sv)
        pltpu.sync_copy(dst_h.at[pl.ds(base, k3)], dv)
        pltpu.sync_copy(e_h.at[pl.ds(base * hn, k3 * hn)], ev)

    def start_gather(sv, rv, sem):
        if esplit:
            return pltpu.async_copy(hh_h.at[sv], rv, sem)
        return pltpu.async_copy(hh_h.at[c].at[sv], rv, sem)

    def mul(rv, ev):
        epq = 16 // hn

        def ibody(q, _):
            evec = ev[pl.ds(q * 16, 16)]
            for r in range(epq):
                i = q * epq + r
                for hh in range(hph):
                    if hn > 1:
                        w = jnp.where(c == 0, evec[r * hn + hh],
                                      evec[r * hn + hph + hh])
                    else:
                        w = evec[r]
                    for g in range(fh // (hph * 16)):
                        off = hh * (fh // hph) + g * 16
                        v = rv[i, pl.ds(off, 16)]
                        rv[i, pl.ds(off, 16)] = v * w
            return 0

        lax.fori_loop(0, k3 // epq, ibody, 0)

    def pbody(t, _):
        ka = 2 * t
        kb = 2 * t + 1
        load_idx(ka, srcv, dstv, eb)
        da = start_gather(srcv, rows_v, gsa)
        load_idx(kb, srcv2, dstv2, eb2)
        db = start_gather(srcv2, rows2, gsb)
        da.wait()
        mul(rows_v, eb)
        dsc = pltpu.async_copy(rows_v, u_sp.at[dstv], ssa, add=True)
        db.wait()
        mul(rows2, eb2)
        dsc.wait()
        pltpu.sync_copy(rows2, u_sp.at[dstv2], add=True)
        return 0

    lax.fori_loop(0, nchunk // 2, pbody, 0)

    if nchunk % 2:
        load_idx(nchunk - 1, srcv, dstv, eb)
        start_gather(srcv, rows_v, gsa).wait()
        mul(rows_v, eb)
        pltpu.sync_copy(rows_v, u_sp.at[dstv], add=True)

    plsc.subcore_barrier()
    nr = NNP // 16
    pltpu.sync_copy(u_sp.at[pl.ds(s * nr, nr)], u_h.at[c, pl.ds(s * nr, nr)])


def _make_e3(hn, hph, fh):
    k3 = 128
    return pl.kernel(
        functools.partial(_e3_body, hn, hph, fh),
        out_type=jax.ShapeDtypeStruct((2, NNP, fh), F32),
        mesh=_sc_mesh(),
        compiler_params=pltpu.CompilerParams(needs_layout_passes=False),
        scratch_types=[
            pltpu.VMEM_SHARED((NNP, fh), F32),
            pltpu.VMEM((k3,), I32),
            pltpu.VMEM((k3,), I32),
            pltpu.VMEM((k3 * hn,), F32),
            pltpu.VMEM((k3, fh), F32),
            pltpu.VMEM((k3,), I32),
            pltpu.VMEM((k3,), I32),
            pltpu.VMEM((k3 * hn,), F32),
            pltpu.VMEM((k3, fh), F32),
            pltpu.SemaphoreType.DMA,
            pltpu.SemaphoreType.DMA,
            pltpu.SemaphoreType.DMA,
        ],
    )


def _d1_body(nin, split, x_refs, w_ref, as_ref, ad_ref, h2_ref, asn_ref,
             adn_ref):
    x = x_refs[0][...]
    for r in x_refs[1:]:
        x = x + r[...]
    h = jnp.dot(x, w_ref[...], preferred_element_type=F32)
    if split:
        fh = h2_ref.shape[2]
        h2_ref[0] = h[:, :fh]
        h2_ref[1] = h[:, fh:]
    else:
        h2_ref[...] = h
    asn_ref[...] = jnp.dot(h, as_ref[...], preferred_element_type=F32)
    adn_ref[...] = jnp.dot(h, ad_ref[...], preferred_element_type=F32)


def _d1(xs, wp, asb, adb, fh, hn):
    nin = len(xs)
    split = hn > 1
    hshape = (2, NNP, fh) if split else (NNP, 2 * fh)

    def body(*refs):
        _d1_body(nin, split, refs[:nin], *refs[nin:])

    return pl.pallas_call(
        body,
        out_shape=(jax.ShapeDtypeStruct(hshape, F32),
                   jax.ShapeDtypeStruct((NNP, hn), F32),
                   jax.ShapeDtypeStruct((NNP, hn), F32)),
    )(*xs, wp, asb, adb)


def _d2_body(mp_ref, m_ref):
    m = jnp.max(mp_ref[...], axis=0)
    m_ref[...] = jnp.where(jnp.isfinite(m), m, 0.0)


def _d2(mpart):
    mw = mpart.shape[1]
    return pl.pallas_call(
        _d2_body,
        out_shape=jax.ShapeDtypeStruct((mw,), F32),
    )(mpart)


def _d3_body(hn, fo, sp_ref, u_ref, b_ref, g_ref, bb_ref, o_ref):
    hph = max(hn // 2, 1)
    chn = fo // hn if hn > 1 else fo
    spn = sp_ref[0] + sp_ref[1]
    if hn == 1:
        xarr = (u_ref[0] + u_ref[1]) / (spn[:, 0:1] + 1e-16)
    else:
        blocks = []
        for c in range(2):
            for hh in range(hph):
                hg = c * hph + hh
                sh = spn[:, hg:hg + 1]
                blk = u_ref[c, :, pl.ds(hh * chn, chn)]
                blocks.append(blk / (sh + 1e-16))
        xarr = jnp.concatenate(blocks, axis=1)
    xb = xarr + b_ref[...]
    valid = lax.broadcasted_iota(I32, (NNP, 1), 0) < N
    xm = jnp.where(valid, xb, 0.0)
    mu = jnp.sum(xm, axis=0, keepdims=True) / N
    d = xb - mu
    var = jnp.sum(jnp.where(valid, d * d, 0.0), axis=0, keepdims=True) / N
    y = g_ref[...] * d / jnp.sqrt(var + 1e-5) + bb_ref[...]
    o_ref[...] = jnp.maximum(y, 0.0)


def _d3(spart, u, b, g, bb, hn, fo):
    sp2 = spart.reshape(2, NNP, hn)
    return pl.pallas_call(
        functools.partial(_d3_body, hn, fo),
        out_shape=jax.ShapeDtypeStruct((NNP, fo), F32),
    )(sp2, u, b.reshape(1, -1), g.reshape(1, -1), bb.reshape(1, -1))


def _pool_body(x4_ref, batch_ref, gf_ref, wg1_ref, bg1_ref, wg2_ref, bg2_ref,
               wp_ref, bp_ref, o_ref):
    x4 = x4_ref[...]
    gate1 = jnp.maximum(
        jnp.dot(x4, wg1_ref[...], preferred_element_type=F32)
        + bg1_ref[...], 0.0)
    gate8 = jnp.dot(gate1, wg2_ref[...], preferred_element_type=F32)
    gate = gate8[:, :1] + bg2_ref[...]
    bcol = batch_ref[...].reshape(N, 1)
    gids = lax.broadcasted_iota(I32, (1, 64), 1)
    onehot = bcol == gids
    gm = jnp.max(jnp.where(onehot, gate, -jnp.inf), axis=0)
    gm = jnp.where(jnp.isfinite(gm), gm, 0.0)
    e64 = jnp.where(onehot, jnp.exp(gate - gm[None, :]), 0.0)
    s64 = jnp.sum(e64, axis=0)
    emb = lax.dot_general(e64, x4, (((0,), (0,)), ((), ())),
                          preferred_element_type=F32)
    emb = emb / (s64[:, None] + 1e-16)
    gf = jnp.maximum(
        jnp.dot(gf_ref[...], wp_ref[...], preferred_element_type=F32)
        + bp_ref[...], 0.0)
    o_ref[:, :128] = emb
    o_ref[:, 128:] = gf


def _pool(x4, batch2d, gfp, wg1, bg1, wg2p, bg2, wpp, bp):
    return pl.pallas_call(
        _pool_body,
        out_shape=jax.ShapeDtypeStruct((64, 160), F32),
    )(x4, batch2d, gfp, wg1, bg1, wg2p, bg2, wpp, bp)


def _blockdiag(a, hn, chn, f):
    out = jnp.zeros((f, hn), F32)
    for h in range(hn):
        out = out.at[h * chn:(h + 1) * chn, h].set(a[h])
    return out


def _layer(x_list, src, dst, ninf4, ninf1, zs4, zs1, zu128, zu64, params, l,
           hn, chn):
    f = hn * chn
    fh = f // 2
    fh3 = f if hn == 1 else f // 2
    hph = max(hn // 2, 1)
    din = params['W%d' % l].shape[0]
    din_p = ((din + 63) // 64) * 64
    wp = jnp.zeros((din_p, f), F32).at[:din].set(params['W%d' % l])
    asb = _blockdiag(params['as%d' % l], hn, chn, f)
    adb = _blockdiag(params['ad%d' % l], hn, chn, f)

    h2, asn, adn = _d1(x_list, wp, asb, adb, fh, hn)
    ninf = ninf4 if hn == 4 else ninf1
    zs = zs4 if hn == 4 else zs1
    zu = zu128
    alpha, mpart = _make_e1(hn)(src, dst, asn.reshape(-1), adn.reshape(-1),
                                ninf)
    m = _d2(mpart.reshape(NW, -1))
    ev, spart = _make_e2(hn)(dst, alpha, m, zs)
    u = _make_e3(hn, hph, fh3)(src, dst, ev, h2, zu)
    xn = _d3(spart, u, params['b%d' % l], params['bng%d' % l],
             params['bnb%d' % l], hn, f)
    return xn


def kernel(x, edge_index, batch, global_features, params):
    loop = jnp.arange(N, dtype=I32)
    padv = jnp.full((E - E0 - N,), N, I32)
    src = jnp.concatenate([edge_index[0], loop, padv])
    dst = jnp.concatenate([edge_index[1], loop, padv])

    ninf4 = jnp.full((NNP * 4,), -3.0e38, F32)
    ninf1 = jnp.full((NNP,), -3.0e38, F32)
    zs4 = jnp.zeros((NNP * 4,), F32)
    zs1 = jnp.zeros((NNP,), F32)
    zu128 = jnp.zeros((NNP, 128), F32)
    zu64 = jnp.zeros((NNP, 64), F32)

    x0 = jnp.zeros((NNP, 64), F32).at[:N, :58].set(x)
    args = (src, dst, ninf4, ninf1, zs4, zs1, zu128, zu64, params)
    x1 = _layer([x0], *args, 0, 4, 64)
    x2 = _layer([x1], *args, 1, 4, 64)
    x3 = _layer([x1, x2], *args, 2, 4, 64)
    x4 = _layer([x3], *args, 3, 1, 128)

    x4s = x4[:N]
    batch2d = batch.reshape(N, 1)
    gfp = jnp.zeros((64, 8), F32).at[:, :7].set(global_features)
    wg2p = jnp.zeros((32, 8), F32).at[:, 0].set(params['Wg2'][:, 0])
    wpp = jnp.zeros((8, 32), F32).at[:7].set(params['Wp'])
    out = _pool(x4s, batch2d, gfp, params['Wg1'],
                params['bg1'].reshape(1, -1), wg2p,
                params['bg2'].reshape(1, 1), wpp,
                params['bp'].reshape(1, -1))
    return out


# speedup vs baseline: 1.2288x; 1.0902x over previous
import functools

import jax
import jax.numpy as jnp
from jax import lax
from jax.experimental import pallas as pl
from jax.experimental.pallas import tpu as pltpu
from jax.experimental.pallas import tpu_sc as plsc

N = 10000
NNP = 10112
E0 = 320000
E = 331776
NW = 32
PW = E // NW
K12 = 3456
PT = E // 16
K3 = 288

F32 = jnp.float32
I32 = jnp.int32

_MESH = dict(core_axis_name="c", subcore_axis_name="s", num_cores=2,
             num_subcores=16)


def _sc_mesh():
    return plsc.VectorSubcoreMesh(**_MESH)


def _e1_body(hn, src_h, dst_h, asn_h, adn_h, ninf_h, alpha_h, mpart_h,
             asn_sp, adn_sp, m_v, srcv, dstv, aidxb, bidxb, ag, bg, ab,
             gs1, gs2):
    c = lax.axis_index("c")
    s = lax.axis_index("s")
    wid = s * 2 + c
    sh = hn.bit_length() - 1
    msk = hn - 1
    kf = K12 * hn

    @pl.when(s == 0)
    def _():
        pltpu.sync_copy(asn_h, asn_sp)
        pltpu.sync_copy(adn_h, adn_sp)

    plsc.subcore_barrier()
    pltpu.sync_copy(ninf_h, m_v)
    lanes = lax.iota(I32, 16)

    for k in range(PW // K12):
        base = wid * PW + k * K12
        pltpu.sync_copy(src_h.at[pl.ds(base, K12)], srcv)
        pltpu.sync_copy(dst_h.at[pl.ds(base, K12)], dstv)

        def idxbody(j, _):
            f = j * 16 + lanes
            e_of = lax.shift_right_logical(f, sh)
            h_of = lax.bitwise_and(f, msk)
            sv = plsc.load_gather(srcv, [e_of])
            dv = plsc.load_gather(dstv, [e_of])
            aidxb[pl.ds(j * 16, 16)] = sv * hn + h_of
            bidxb[pl.ds(j * 16, 16)] = dv * hn + h_of
            return 0

        lax.fori_loop(0, kf // 16, idxbody, 0)
        dga = pltpu.async_copy(asn_sp.at[aidxb], ag, gs1)
        dgb = pltpu.async_copy(adn_sp.at[bidxb], bg, gs2)
        dga.wait()
        dgb.wait()

        def jbody(j, _):
            al = ag[pl.ds(j * 16, 16)] + bg[pl.ds(j * 16, 16)]
            al = jnp.where(al >= 0.0, al, 0.2 * al)
            ab[pl.ds(j * 16, 16)] = al
            midx = bidxb[pl.ds(j * 16, 16)]
            cur = plsc.load_gather(m_v, [midx])

            def wcond(cu):
                return jnp.any(al > cu)

            def wbody(cu):
                plsc.store_scatter(m_v, [midx], al, mask=al > cu)
                return plsc.load_gather(m_v, [midx])

            lax.while_loop(wcond, wbody, cur)
            return 0

        lax.fori_loop(0, kf // 16, jbody, 0)
        pltpu.sync_copy(ab, alpha_h.at[pl.ds(base * hn, kf)])

    pltpu.sync_copy(m_v, mpart_h.at[pl.ds(wid * (NNP * hn), NNP * hn)])


def _make_e1(hn):
    mw = NNP * hn
    kf = K12 * hn
    return pl.kernel(
        functools.partial(_e1_body, hn),
        out_type=(jax.ShapeDtypeStruct((E * hn,), F32),
                  jax.ShapeDtypeStruct((NW * mw,), F32)),
        mesh=_sc_mesh(),
        compiler_params=pltpu.CompilerParams(needs_layout_passes=False),
        scratch_types=[
            pltpu.VMEM_SHARED((mw,), F32),
            pltpu.VMEM_SHARED((mw,), F32),
            pltpu.VMEM((mw,), F32),
            pltpu.VMEM((K12,), I32),
            pltpu.VMEM((K12,), I32),
            pltpu.VMEM((kf,), I32),
            pltpu.VMEM((kf,), I32),
            pltpu.VMEM((kf,), F32),
            pltpu.VMEM((kf,), F32),
            pltpu.VMEM((kf,), F32),
            pltpu.SemaphoreType.DMA,
            pltpu.SemaphoreType.DMA,
        ],
    )


def _e2_body(hn, dst_h, alpha_h, m_h, zs_h, e_h, spart_h,
             m_sp, s_sp, dstv, bidxb, avb, mg, eb, gs1):
    c = lax.axis_index("c")
    s = lax.axis_index("s")
    wid = s * 2 + c
    sh = hn.bit_length() - 1
    msk = hn - 1
    kf = K12 * hn

    @pl.when(s == 0)
    def _():
        pltpu.sync_copy(m_h, m_sp)
        pltpu.sync_copy(zs_h, s_sp)

    plsc.subcore_barrier()
    lanes = lax.iota(I32, 16)

    for k in range(PW // K12):
        base = wid * PW + k * K12
        pltpu.sync_copy(dst_h.at[pl.ds(base, K12)], dstv)
        dav = pltpu.async_copy(alpha_h.at[pl.ds(base * hn, kf)], avb, gs1)

        def idxbody(j, _):
            f = j * 16 + lanes
            e_of = lax.shift_right_logical(f, sh)
            h_of = lax.bitwise_and(f, msk)
            dv = plsc.load_gather(dstv, [e_of])
            bidxb[pl.ds(j * 16, 16)] = dv * hn + h_of
            return 0

        lax.fori_loop(0, kf // 16, idxbody, 0)
        pltpu.sync_copy(m_sp.at[bidxb], mg)
        dav.wait()

        def jbody(j, _):
            ev = jnp.exp(avb[pl.ds(j * 16, 16)] - mg[pl.ds(j * 16, 16)])
            eb[pl.ds(j * 16, 16)] = ev
            return 0

        lax.fori_loop(0, kf // 16, jbody, 0)
        pltpu.sync_copy(eb, e_h.at[pl.ds(base * hn, kf)])
        pltpu.sync_copy(eb, s_sp.at[bidxb], add=True)

    plsc.subcore_barrier()

    @pl.when(s == 0)
    def _():
        pltpu.sync_copy(s_sp, spart_h.at[pl.ds(c * (NNP * hn), NNP * hn)])


def _make_e2(hn):
    mw = NNP * hn
    kf = K12 * hn
    return pl.kernel(
        functools.partial(_e2_body, hn),
        out_type=(jax.ShapeDtypeStruct((E * hn,), F32),
                  jax.ShapeDtypeStruct((2 * mw,), F32)),
        mesh=_sc_mesh(),
        compiler_params=pltpu.CompilerParams(needs_layout_passes=False),
        scratch_types=[
            pltpu.VMEM_SHARED((mw,), F32),
            pltpu.VMEM_SHARED((mw,), F32),
            pltpu.VMEM((K12,), I32),
            pltpu.VMEM((kf,), I32),
            pltpu.VMEM((kf,), F32),
            pltpu.VMEM((kf,), F32),
            pltpu.VMEM((kf,), F32),
            pltpu.SemaphoreType.DMA,
        ],
    )


def _e3_body(hn, hph, fh, src_h, dst_h, e_h, hh_h, zu_h, u_h,
             u_sp, srcv, dstv, eb, rows_v, srcv2, dstv2, eb2, rows2,
             gsa, gsb, ssa):
    c = lax.axis_index("c")
    s = lax.axis_index("s")
    esplit = hn == 1
    k3 = 128
    nchunk = (E // NW) // k3 if esplit else PT // k3

    @pl.when(s == 0)
    def _():
        pltpu.sync_copy(zu_h, u_sp)

    plsc.subcore_barrier()

    def chunk_base(k):
        if esplit:
            return (s * 2 + c) * (E // NW) + k * k3
        return s * PT + k * k3

    def load_idx(k, sv, dv, ev):
        base = chunk_base(k)
        pltpu.sync_copy(src_h.at[pl.ds(base, k3)], sv)
        pltpu.sync_copy(dst_h.at[pl.ds(base, k3)], dv)
        pltpu.sync_copy(e_h.at[pl.ds(base * hn, k3 * hn)], ev)

    def start_gather(sv, rv, sem):
        if esplit:
            return pltpu.async_copy(hh_h.at[sv], rv, sem)
        return pltpu.async_copy(hh_h.at[c].at[sv], rv, sem)

    def mul(rv, ev):
        epq = 16 // hn

        def ibody(q, _):
            evec = ev[pl.ds(q * 16, 16)]
            for r in range(epq):
                i = q * epq + r
                for hh in range(hph):
                    if hn > 1:
                        w = jnp.where(c == 0, evec[r * hn + hh],
                                      evec[r * hn + hph + hh])
                    else:
                        w = evec[r]
                    for g in range(fh // (hph * 16)):
                        off = hh * (fh // hph) + g * 16
                        v = rv[i, pl.ds(off, 16)]
                        rv[i, pl.ds(off, 16)] = v * w
            return 0

        lax.fori_loop(0, k3 // epq, ibody, 0)

    def pbody(t, _):
        ka = 2 * t
        kb = 2 * t + 1
        load_idx(ka, srcv, dstv, eb)
        da = start_gather(srcv, rows_v, gsa)
        load_idx(kb, srcv2, dstv2, eb2)
        db = start_gather(srcv2, rows2, gsb)
        da.wait()
        mul(rows_v, eb)
        dsc = pltpu.async_copy(rows_v, u_sp.at[dstv], ssa, add=True)
        db.wait()
        mul(rows2, eb2)
        dsc.wait()
        pltpu.sync_copy(rows2, u_sp.at[dstv2], add=True)
        return 0

    lax.fori_loop(0, nchunk // 2, pbody, 0)

    if nchunk % 2:
        load_idx(nchunk - 1, srcv, dstv, eb)
        start_gather(srcv, rows_v, gsa).wait()
        mul(rows_v, eb)
        pltpu.sync_copy(rows_v, u_sp.at[dstv], add=True)

    plsc.subcore_barrier()
    nr = NNP // 16
    pltpu.sync_copy(u_sp.at[pl.ds(s * nr, nr)], u_h.at[c, pl.ds(s * nr, nr)])


def _make_e3(hn, hph, fh):
    k3 = 128
    return pl.kernel(
        functools.partial(_e3_body, hn, hph, fh),
        out_type=jax.ShapeDtypeStruct((2, NNP, fh), F32),
        mesh=_sc_mesh(),
        compiler_params=pltpu.CompilerParams(needs_layout_passes=False),
        scratch_types=[
            pltpu.VMEM_SHARED((NNP, fh), F32),
            pltpu.VMEM((k3,), I32),
            pltpu.VMEM((k3,), I32),
            pltpu.VMEM((k3 * hn,), F32),
            pltpu.VMEM((k3, fh), F32),
            pltpu.VMEM((k3,), I32),
            pltpu.VMEM((k3,), I32),
            pltpu.VMEM((k3 * hn,), F32),
            pltpu.VMEM((k3, fh), F32),
            pltpu.SemaphoreType.DMA,
            pltpu.SemaphoreType.DMA,
            pltpu.SemaphoreType.DMA,
        ],
    )


def _d1_body(nin, split, x_refs, w_ref, as_ref, ad_ref, h2_ref, asn_ref,
             adn_ref):
    x = x_refs[0][...]
    for r in x_refs[1:]:
        x = x + r[...]
    h = jnp.dot(x, w_ref[...], preferred_element_type=F32)
    if split:
        fh = h2_ref.shape[2]
        h2_ref[0] = h[:, :fh]
        h2_ref[1] = h[:, fh:]
    else:
        h2_ref[...] = h
    asn_ref[...] = jnp.dot(h, as_ref[...], preferred_element_type=F32)
    adn_ref[...] = jnp.dot(h, ad_ref[...], preferred_element_type=F32)


def _d1(xs, wp, asb, adb, fh, hn):
    nin = len(xs)
    split = hn > 1
    hshape = (2, NNP, fh) if split else (NNP, 2 * fh)

    def body(*refs):
        _d1_body(nin, split, refs[:nin], *refs[nin:])

    return pl.pallas_call(
        body,
        out_shape=(jax.ShapeDtypeStruct(hshape, F32),
                   jax.ShapeDtypeStruct((NNP, hn), F32),
                   jax.ShapeDtypeStruct((NNP, hn), F32)),
    )(*xs, wp, asb, adb)


def _d2_body(mp_ref, m_ref):
    m = jnp.max(mp_ref[...], axis=0)
    m_ref[...] = jnp.where(jnp.isfinite(m), m, 0.0)


def _d2(mpart):
    mw = mpart.shape[1]
    return pl.pallas_call(
        _d2_body,
        out_shape=jax.ShapeDtypeStruct((mw,), F32),
    )(mpart)


def _d3_body(hn, fo, sp_ref, u_ref, b_ref, g_ref, bb_ref, o_ref):
    hph = max(hn // 2, 1)
    chn = fo // hn if hn > 1 else fo
    spn = sp_ref[0] + sp_ref[1]
    if hn == 1:
        xarr = (u_ref[0] + u_ref[1]) / (spn[:, 0:1] + 1e-16)
    else:
        blocks = []
        for c in range(2):
            for hh in range(hph):
                hg = c * hph + hh
                sh = spn[:, hg:hg + 1]
                blk = u_ref[c, :, pl.ds(hh * chn, chn)]
                blocks.append(blk / (sh + 1e-16))
        xarr = jnp.concatenate(blocks, axis=1)
    xb = xarr + b_ref[...]
    valid = lax.broadcasted_iota(I32, (NNP, 1), 0) < N
    xm = jnp.where(valid, xb, 0.0)
    mu = jnp.sum(xm, axis=0, keepdims=True) / N
    d = xb - mu
    var = jnp.sum(jnp.where(valid, d * d, 0.0), axis=0, keepdims=True) / N
    y = g_ref[...] * d / jnp.sqrt(var + 1e-5) + bb_ref[...]
    o_ref[...] = jnp.maximum(y, 0.0)


def _d3(spart, u, b, g, bb, hn, fo):
    sp2 = spart.reshape(2, NNP, hn)
    return pl.pallas_call(
        functools.partial(_d3_body, hn, fo),
        out_shape=jax.ShapeDtypeStruct((NNP, fo), F32),
    )(sp2, u, b.reshape(1, -1), g.reshape(1, -1), bb.reshape(1, -1))


def _pool_body(x4_ref, batch_ref, gf_ref, wg1_ref, bg1_ref, wg2_ref, bg2_ref,
               wp_ref, bp_ref, o_ref):
    x4 = x4_ref[...]
    gate1 = jnp.maximum(
        jnp.dot(x4, wg1_ref[...], preferred_element_type=F32)
        + bg1_ref[...], 0.0)
    gate8 = jnp.dot(gate1, wg2_ref[...], preferred_element_type=F32)
    gate = gate8[:, :1] + bg2_ref[...]
    bcol = batch_ref[...].reshape(N, 1)
    gids = lax.broadcasted_iota(I32, (1, 64), 1)
    onehot = bcol == gids
    gm = jnp.max(jnp.where(onehot, gate, -jnp.inf), axis=0)
    gm = jnp.where(jnp.isfinite(gm), gm, 0.0)
    e64 = jnp.where(onehot, jnp.exp(gate - gm[None, :]), 0.0)
    s64 = jnp.sum(e64, axis=0)
    emb = lax.dot_general(e64, x4, (((0,), (0,)), ((), ())),
                          preferred_element_type=F32)
    emb = emb / (s64[:, None] + 1e-16)
    gf = jnp.maximum(
        jnp.dot(gf_ref[...], wp_ref[...], preferred_element_type=F32)
        + bp_ref[...], 0.0)
    o_ref[:, :128] = emb
    o_ref[:, 128:] = gf


def _pool(x4, batch2d, gfp, wg1, bg1, wg2p, bg2, wpp, bp):
    return pl.pallas_call(
        _pool_body,
        out_shape=jax.ShapeDtypeStruct((64, 160), F32),
    )(x4, batch2d, gfp, wg1, bg1, wg2p, bg2, wpp, bp)


def _blockdiag(a, hn, chn, f):
    out = jnp.zeros((f, hn), F32)
    for h in range(hn):
        out = out.at[h * chn:(h + 1) * chn, h].set(a[h])
    return out


def _layer(x_list, src, dst, ninf4, ninf1, zs4, zs1, zu128, zu64, params, l,
           hn, chn):
    f = hn * chn
    fh = f // 2
    fh3 = f if hn == 1 else f // 2
    hph = max(hn // 2, 1)
    din = params['W%d' % l].shape[0]
    din_p = ((din + 63) // 64) * 64
    wp = jnp.zeros((din_p, f), F32).at[:din].set(params['W%d' % l])
    asb = _blockdiag(params['as%d' % l], hn, chn, f)
    adb = _blockdiag(params['ad%d' % l], hn, chn, f)

    h2, asn, adn = _d1(x_list, wp, asb, adb, fh, hn)
    ninf = ninf4 if hn == 4 else ninf1
    zs = zs4 if hn == 4 else zs1
    zu = zu128
    alpha, mpart = _make_e1(hn)(src, dst, asn.reshape(-1), adn.reshape(-1),
                                ninf)
    m = _d2(mpart.reshape(NW, -1))
    ev, spart = _make_e2(hn)(dst, alpha, m, zs)
    u = _make_e3(hn, hph, fh3)(src, dst, ev, h2, zu)
    xn = _d3(spart, u, params['b%d' % l], params['bng%d' % l],
             params['bnb%d' % l], hn, f)
    return xn


def kernel(x, edge_index, batch, global_features, params):
    loop = jnp.arange(N, dtype=I32)
    padv = N + (jnp.arange(E - E0 - N, dtype=I32) % (NNP - N))
    src = jnp.concatenate([edge_index[0], loop, padv])
    dst = jnp.concatenate([edge_index[1], loop, padv])

    ninf4 = jnp.full((NNP * 4,), -3.0e38, F32)
    ninf1 = jnp.full((NNP,), -3.0e38, F32)
    zs4 = jnp.zeros((NNP * 4,), F32)
    zs1 = jnp.zeros((NNP,), F32)
    zu128 = jnp.zeros((NNP, 128), F32)
    zu64 = jnp.zeros((NNP, 64), F32)

    x0 = jnp.zeros((NNP, 64), F32).at[:N, :58].set(x)
    args = (src, dst, ninf4, ninf1, zs4, zs1, zu128, zu64, params)
    x1 = _layer([x0], *args, 0, 4, 64)
    x2 = _layer([x1], *args, 1, 4, 64)
    x3 = _layer([x1, x2], *args, 2, 4, 64)
    x4 = _layer([x3], *args, 3, 1, 128)

    x4s = x4[:N]
    batch2d = batch.reshape(N, 1)
    gfp = jnp.zeros((64, 8), F32).at[:, :7].set(global_features)
    wg2p = jnp.zeros((32, 8), F32).at[:, 0].set(params['Wg2'][:, 0])
    wpp = jnp.zeros((8, 32), F32).at[:7].set(params['Wp'])
    out = _pool(x4s, batch2d, gfp, params['Wg1'],
                params['bg1'].reshape(1, -1), wg2p,
                params['bg2'].reshape(1, 1), wpp,
                params['bp'].reshape(1, -1))
    return out
